# Initial kernel scaffold; baseline (speedup 1.0000x reference)
import jax
import jax.numpy as jnp
from jax.experimental import pallas as pl


def kernel(node_attr, edge_attr, edge_index, params):
    raise NotImplementedError("write your pallas kernel here")


# speedup vs baseline: 1.8215x; 1.8215x over previous
import functools

import jax
import jax.numpy as jnp
from jax import lax
from jax.experimental import pallas as pl
from jax.experimental.pallas import tpu as pltpu
from jax.experimental.pallas import tpu_sc as plsc

HID = 64
N_FINE = 10000
NP = 10240
E = 160000
EP = 163840
NC = 5000
NCP = 5120
ACC_F = 10256
ACC_C = 5136
NW = 32
EPW = EP // NW
LN_EPS = 1e-5


def _ln(h, g, be):
    mu = jnp.mean(h, axis=-1, keepdims=True)
    var = jnp.mean((h - mu) ** 2, axis=-1, keepdims=True)
    return (h - mu) * lax.rsqrt(var + LN_EPS) * g + be


def _enc(x, p, block):
    n, din = x.shape

    def body(x_ref, w1_ref, b1_ref, w2_ref, b2_ref, g_ref, be_ref, o_ref):
        h = jnp.dot(x_ref[...], w1_ref[...], preferred_element_type=jnp.float32)
        h = jnp.maximum(h + b1_ref[...], 0.0)
        h = jnp.dot(h, w2_ref[...], preferred_element_type=jnp.float32) + b2_ref[...]
        o_ref[...] = _ln(h, g_ref[...], be_ref[...])

    return pl.pallas_call(
        body,
        grid=(n // block,),
        in_specs=[
            pl.BlockSpec((block, din), lambda i: (i, 0)),
            pl.BlockSpec((din, HID), lambda i: (0, 0)),
            pl.BlockSpec((1, HID), lambda i: (0, 0)),
            pl.BlockSpec((HID, HID), lambda i: (0, 0)),
            pl.BlockSpec((1, HID), lambda i: (0, 0)),
            pl.BlockSpec((1, HID), lambda i: (0, 0)),
            pl.BlockSpec((1, HID), lambda i: (0, 0)),
        ],
        out_specs=pl.BlockSpec((block, HID), lambda i: (i, 0)),
        out_shape=jax.ShapeDtypeStruct((n, HID), jnp.float32),
    )(x, p['W1'], p['b1'].reshape(1, HID), p['W2'], p['b2'].reshape(1, HID),
      p['g'].reshape(1, HID), p['be'].reshape(1, HID))


def _edge_mlp(gxr, gxc, e, p, cnt=None, block=2048):
    n = gxr.shape[0]
    has_cnt = cnt is not None

    def body(*refs):
        if has_cnt:
            (a_ref, b_ref, e_ref, cnt_ref, w1_ref, b1_ref, w2_ref, b2_ref,
             g_ref, be_ref, o_ref) = refs
            base = e_ref[...] * (1.0 / jnp.maximum(cnt_ref[...], 1.0))
        else:
            (a_ref, b_ref, e_ref, w1_ref, b1_ref, w2_ref, b2_ref,
             g_ref, be_ref, o_ref) = refs
            base = e_ref[...]
        w1 = w1_ref[...]
        h = (jnp.dot(a_ref[...], w1[0:HID], preferred_element_type=jnp.float32)
             + jnp.dot(b_ref[...], w1[HID:2 * HID], preferred_element_type=jnp.float32)
             + jnp.dot(base, w1[2 * HID:3 * HID], preferred_element_type=jnp.float32)
             + b1_ref[...])
        h = jnp.maximum(h, 0.0)
        h = jnp.dot(h, w2_ref[...], preferred_element_type=jnp.float32) + b2_ref[...]
        o_ref[...] = base + _ln(h, g_ref[...], be_ref[...])

    row_spec = pl.BlockSpec((block, HID), lambda i: (i, 0))
    in_specs = [row_spec, row_spec, row_spec]
    args = [gxr, gxc, e]
    if has_cnt:
        in_specs.append(pl.BlockSpec((block, 1), lambda i: (i, 0)))
        args.append(cnt)
    in_specs += [
        pl.BlockSpec((3 * HID, HID), lambda i: (0, 0)),
        pl.BlockSpec((1, HID), lambda i: (0, 0)),
        pl.BlockSpec((HID, HID), lambda i: (0, 0)),
        pl.BlockSpec((1, HID), lambda i: (0, 0)),
        pl.BlockSpec((1, HID), lambda i: (0, 0)),
        pl.BlockSpec((1, HID), lambda i: (0, 0)),
    ]
    args += [p['W1'], p['b1'].reshape(1, HID), p['W2'], p['b2'].reshape(1, HID),
             p['g'].reshape(1, HID), p['be'].reshape(1, HID)]
    return pl.pallas_call(
        body,
        grid=(n // block,),
        in_specs=in_specs,
        out_specs=row_spec,
        out_shape=jax.ShapeDtypeStruct((n, HID), jnp.float32),
    )(*args)


def _node_mlp_body(x_ref, a_ref, b_ref, w1_ref, b1_ref, w2_ref, b2_ref,
                   g_ref, be_ref):
    agg = a_ref[...] + b_ref[...]
    w1 = w1_ref[...]
    h = (jnp.dot(x_ref[...], w1[0:HID], preferred_element_type=jnp.float32)
         + jnp.dot(agg, w1[HID:2 * HID], preferred_element_type=jnp.float32)
         + b1_ref[...])
    h = jnp.maximum(h, 0.0)
    h = jnp.dot(h, w2_ref[...], preferred_element_type=jnp.float32) + b2_ref[...]
    return x_ref[...] + _ln(h, g_ref[...], be_ref[...])


def _node_specs(block):
    row_spec = pl.BlockSpec((block, HID), lambda i: (i, 0))
    return [
        row_spec, row_spec, row_spec,
        pl.BlockSpec((2 * HID, HID), lambda i: (0, 0)),
        pl.BlockSpec((1, HID), lambda i: (0, 0)),
        pl.BlockSpec((HID, HID), lambda i: (0, 0)),
        pl.BlockSpec((1, HID), lambda i: (0, 0)),
        pl.BlockSpec((1, HID), lambda i: (0, 0)),
        pl.BlockSpec((1, HID), lambda i: (0, 0)),
    ]


def _node_args(x, agg_a, agg_b, p):
    return [x, agg_a, agg_b, p['W1'], p['b1'].reshape(1, HID), p['W2'],
            p['b2'].reshape(1, HID), p['g'].reshape(1, HID), p['be'].reshape(1, HID)]


def _node_mlp(x, agg_a, agg_b, p, block=2048):
    n = x.shape[0]

    def body(x_ref, a_ref, b_ref, w1_ref, b1_ref, w2_ref, b2_ref, g_ref,
             be_ref, o_ref):
        o_ref[...] = _node_mlp_body(x_ref, a_ref, b_ref, w1_ref, b1_ref,
                                    w2_ref, b2_ref, g_ref, be_ref)

    return pl.pallas_call(
        body,
        grid=(n // block,),
        in_specs=_node_specs(block),
        out_specs=pl.BlockSpec((block, HID), lambda i: (i, 0)),
        out_shape=jax.ShapeDtypeStruct((n, HID), jnp.float32),
    )(*_node_args(x, agg_a, agg_b, p))


def _node_mlp_unpool(x, agg_a, agg_b, p, skip3, block=1024):
    n = x.shape[0]

    def body(x_ref, a_ref, b_ref, w1_ref, b1_ref, w2_ref, b2_ref, g_ref,
             be_ref, s_ref, o_ref):
        x_new = _node_mlp_body(x_ref, a_ref, b_ref, w1_ref, b1_ref, w2_ref,
                               b2_ref, g_ref, be_ref)
        o_ref[...] = x_new[:, None, :] + s_ref[...]

    return pl.pallas_call(
        body,
        grid=(n // block,),
        in_specs=_node_specs(block) + [pl.BlockSpec((block, 2, HID), lambda i: (i, 0, 0))],
        out_specs=pl.BlockSpec((block, 2, HID), lambda i: (i, 0, 0)),
        out_shape=jax.ShapeDtypeStruct((n, 2, HID), jnp.float32),
    )(*_node_args(x, agg_a, agg_b, p), skip3)


def _node_mlp_dec(x, agg_a, agg_b, p, pdec, block=2048):
    n = x.shape[0]

    def body(x_ref, a_ref, b_ref, w1_ref, b1_ref, w2_ref, b2_ref, g_ref,
             be_ref, d1_ref, db1_ref, d2_ref, db2_ref, o_ref):
        x_new = _node_mlp_body(x_ref, a_ref, b_ref, w1_ref, b1_ref, w2_ref,
                               b2_ref, g_ref, be_ref)
        h = jnp.dot(x_new, d1_ref[...], preferred_element_type=jnp.float32)
        h = jnp.maximum(h + db1_ref[...], 0.0)
        o_ref[...] = jnp.dot(h, d2_ref[...], preferred_element_type=jnp.float32) + db2_ref[...]

    return pl.pallas_call(
        body,
        grid=(n // block,),
        in_specs=_node_specs(block) + [
            pl.BlockSpec((HID, HID), lambda i: (0, 0)),
            pl.BlockSpec((1, HID), lambda i: (0, 0)),
            pl.BlockSpec((HID, 3), lambda i: (0, 0)),
            pl.BlockSpec((1, 3), lambda i: (0, 0)),
        ],
        out_specs=pl.BlockSpec((block, 3), lambda i: (i, 0)),
        out_shape=jax.ShapeDtypeStruct((n, 3), jnp.float32),
    )(*_node_args(x, agg_a, agg_b, p), pdec['W1'], pdec['b1'].reshape(1, HID),
      pdec['W2'], pdec['b2'].reshape(1, 3))


def _pool(x3, block=1024):
    n = x3.shape[0]

    def body(x_ref, o_ref):
        o_ref[...] = 0.5 * (x_ref[:, 0, :] + x_ref[:, 1, :])

    return pl.pallas_call(
        body,
        grid=(n // block,),
        in_specs=[pl.BlockSpec((block, 2, HID), lambda i: (i, 0, 0))],
        out_specs=pl.BlockSpec((block, HID), lambda i: (i, 0)),
        out_shape=jax.ShapeDtypeStruct((n, HID), jnp.float32),
    )(x3)


@functools.cache
def _mesh():
    return plsc.VectorSubcoreMesh(core_axis_name="c", subcore_axis_name="s")


@functools.cache
def _make_gather_pair():
    @functools.partial(
        pl.kernel, mesh=_mesh(),
        out_type=[jax.ShapeDtypeStruct((EP, HID), jnp.float32),
                  jax.ShapeDtypeStruct((EP, HID), jnp.float32)],
        scratch_types=[pltpu.VMEM((EPW // 128, 128), jnp.int32),
                       pltpu.VMEM((1024, HID), jnp.float32),
                       pltpu.SemaphoreType.DMA],
        compiler_params=pltpu.CompilerParams(use_tc_tiling_on_sc=False),
    )
    def k(x_hbm, idxa_hbm, idxb_hbm, outa_hbm, outb_hbm, idx_v, rows_v, sem):
        wid = lax.axis_index("s") * 2 + lax.axis_index("c")
        base = wid * EPW
        for idx_hbm, out_hbm in ((idxa_hbm, outa_hbm), (idxb_hbm, outb_hbm)):
            pltpu.sync_copy(idx_hbm.at[wid], idx_v)

            def g_body(g, carry):
                cps = [pltpu.async_copy(x_hbm.at[idx_v.at[g * 8 + j]],
                                        rows_v.at[pl.ds(j * 128, 128)], sem)
                       for j in range(8)]
                for cp in cps:
                    cp.wait()
                pltpu.sync_copy(rows_v, out_hbm.at[pl.ds(base + g * 1024, 1024)])
                return carry

            lax.fori_loop(0, EPW // 1024, g_body, 0)

    return k


def _sc_gather_pair(x, idxa, idxb):
    return _make_gather_pair()(x, idxa, idxb)


@functools.cache
def _make_scatter(acc_rows):
    rpt = acc_rows // 16

    @functools.partial(
        pl.kernel, mesh=_mesh(),
        out_type=jax.ShapeDtypeStruct((2, acc_rows, HID), jnp.float32),
        scratch_types=[pltpu.VMEM_SHARED((acc_rows, HID), jnp.float32),
                       pltpu.VMEM((EPW // 128, 128), jnp.int32),
                       pltpu.VMEM((512, HID), jnp.float32)],
        compiler_params=pltpu.CompilerParams(use_tc_tiling_on_sc=False),
    )
    def k(vals_hbm, idx_hbm, zeros_hbm, out_hbm, acc_s, idx_v, ebuf):
        cid = lax.axis_index("c")
        sid = lax.axis_index("s")
        wid = sid * 2 + cid
        base = wid * EPW
        pltpu.sync_copy(zeros_hbm.at[pl.ds(sid * rpt, rpt)],
                        acc_s.at[pl.ds(sid * rpt, rpt)])
        plsc.subcore_barrier()
        pltpu.sync_copy(idx_hbm.at[wid], idx_v)

        def g_body(g, carry):
            pltpu.sync_copy(vals_hbm.at[pl.ds(base + g * 512, 512)], ebuf)
            for j in range(4):
                pltpu.sync_copy(ebuf.at[pl.ds(j * 128, 128)],
                                acc_s.at[idx_v.at[g * 4 + j]], add=True)
            return carry

        lax.fori_loop(0, EPW // 512, g_body, 0)
        plsc.subcore_barrier()
        pltpu.sync_copy(acc_s.at[pl.ds(sid * rpt, rpt)],
                        out_hbm.at[cid, pl.ds(sid * rpt, rpt)])

    return k


def _sc_scatter_fine(vals, idx, zeros):
    return _make_scatter(ACC_F)(vals, idx, zeros)


def _sc_scatter_coarse(vals, idx, zeros):
    return _make_scatter(ACC_C)(vals, idx, zeros)


def _r3(a):
    return a.astype(jnp.int32).reshape(NW, EPW // 128, 128)


def kernel(node_attr, edge_attr, edge_index, params):
    f32 = jnp.float32
    p = params
    eid = jnp.arange(EP, dtype=jnp.int32)
    real = eid < E

    nap = jnp.pad(node_attr.astype(f32), ((0, NP - N_FINE), (0, 0)))
    eap = jnp.pad(edge_attr.astype(f32), ((0, EP - E), (0, 0)))
    rowp = jnp.pad(edge_index[0].astype(jnp.int32), (0, EP - E))
    colp = jnp.pad(edge_index[1].astype(jnp.int32), (0, EP - E))
    col_s = jnp.where(real, colp, NP)
    crow = rowp // 2
    ccol = colp // 2

    keys = jnp.where(real, crow * NC + ccol, NC * NC)
    table = jnp.zeros((NC * NC + 1,), jnp.int32).at[keys].set(eid)
    rep = table[keys]
    is_rep = (rep == eid) & real
    ccol_s = jnp.where(is_rep, ccol, NCP)
    cnt = jax.ops.segment_sum(jnp.ones((EP,), f32), rep, num_segments=EP)
    cnt2 = cnt.reshape(EP, 1)

    idx_row = _r3(rowp)
    idx_col = _r3(colp)
    idx_col_s = _r3(col_s)
    idx_crow = _r3(crow)
    idx_ccol = _r3(ccol)
    idx_ccol_s = _r3(ccol_s)
    zeros_f = jnp.zeros((ACC_F, HID), f32)
    zeros_c = jnp.zeros((ACC_C, HID), f32)

    x0 = _enc(nap, p['node_enc'], block=2048)
    e0 = _enc(eap, p['edge_enc'], block=2048)

    lp = p['down'][0]
    gxr, gxc = _sc_gather_pair(x0, idx_row, idx_col)
    e1 = _edge_mlp(gxr, gxc, e0, lp['edge'])
    parts = _sc_scatter_fine(e1, idx_col_s, zeros_f)
    x1 = _node_mlp(x0, parts[0, :NP], parts[1, :NP], lp['node'], block=2048)

    cx = _pool(x1.reshape(NCP, 2, HID))
    ce_sums = jax.ops.segment_sum(e1, rep, num_segments=EP)

    lp = p['bottleneck'][0]
    cgr, cgc = _sc_gather_pair(cx, idx_crow, idx_ccol)
    ec = _edge_mlp(cgr, cgc, ce_sums, lp['edge'], cnt=cnt2)
    cparts = _sc_scatter_coarse(ec, idx_ccol_s, zeros_c)
    cx = _node_mlp(cx, cparts[0, :NCP], cparts[1, :NCP], lp['node'], block=1024)

    lp = p['bottleneck'][1]
    cgr, cgc = _sc_gather_pair(cx, idx_crow, idx_ccol)
    ec = _edge_mlp(cgr, cgc, ec, lp['edge'])
    cparts = _sc_scatter_coarse(ec, idx_ccol_s, zeros_c)
    xup3 = _node_mlp_unpool(cx, cparts[0, :NCP], cparts[1, :NCP], lp['node'],
                            x1.reshape(NCP, 2, HID), block=1024)
    xup = xup3.reshape(NP, HID)

    lp = p['up'][0]
    gxr, gxc = _sc_gather_pair(xup, idx_row, idx_col)
    eu = _edge_mlp(gxr, gxc, e1, lp['edge'])
    parts = _sc_scatter_fine(eu, idx_col_s, zeros_f)
    out = _node_mlp_dec(xup, parts[0, :NP], parts[1, :NP], lp['node'],
                        p['dec'], block=2048)
    return out[:N_FINE]


# speedup vs baseline: 2.1410x; 1.1754x over previous
import functools

import jax
import jax.numpy as jnp
from jax import lax
from jax.experimental import pallas as pl
from jax.experimental.pallas import tpu as pltpu
from jax.experimental.pallas import tpu_sc as plsc

HID = 64
N_FINE = 10000
NP = 10240
E = 160000
EP = 163840
NC = 5000
NCP = 5120
ACC_F = 10256
ACC_C = 5136
NW = 32
EPW = EP // NW
LN_EPS = 1e-5


def _ln(h, g, be):
    mu = jnp.mean(h, axis=-1, keepdims=True)
    var = jnp.mean((h - mu) ** 2, axis=-1, keepdims=True)
    return (h - mu) * lax.rsqrt(var + LN_EPS) * g + be


def _enc(x, p, block):
    n, din = x.shape

    def body(x_ref, w1_ref, b1_ref, w2_ref, b2_ref, g_ref, be_ref, o_ref):
        h = jnp.dot(x_ref[...], w1_ref[...], preferred_element_type=jnp.float32)
        h = jnp.maximum(h + b1_ref[...], 0.0)
        h = jnp.dot(h, w2_ref[...], preferred_element_type=jnp.float32) + b2_ref[...]
        o_ref[...] = _ln(h, g_ref[...], be_ref[...])

    return pl.pallas_call(
        body,
        grid=(n // block,),
        in_specs=[
            pl.BlockSpec((block, din), lambda i: (i, 0)),
            pl.BlockSpec((din, HID), lambda i: (0, 0)),
            pl.BlockSpec((1, HID), lambda i: (0, 0)),
            pl.BlockSpec((HID, HID), lambda i: (0, 0)),
            pl.BlockSpec((1, HID), lambda i: (0, 0)),
            pl.BlockSpec((1, HID), lambda i: (0, 0)),
            pl.BlockSpec((1, HID), lambda i: (0, 0)),
        ],
        out_specs=pl.BlockSpec((block, HID), lambda i: (i, 0)),
        out_shape=jax.ShapeDtypeStruct((n, HID), jnp.float32),
    )(x, p['W1'], p['b1'].reshape(1, HID), p['W2'], p['b2'].reshape(1, HID),
      p['g'].reshape(1, HID), p['be'].reshape(1, HID))


def _edge_mlp(gxr, gxc, e, p, cnt=None, block=2048):
    n = gxr.shape[0]
    has_cnt = cnt is not None

    def body(*refs):
        if has_cnt:
            (a_ref, b_ref, e_ref, ca_ref, cb_ref, w1_ref, b1_ref, w2_ref,
             b2_ref, g_ref, be_ref, o_ref) = refs
            base = e_ref[...] * (1.0 / jnp.maximum(ca_ref[...] + cb_ref[...],
                                                   1.0))
        else:
            (a_ref, b_ref, e_ref, w1_ref, b1_ref, w2_ref, b2_ref,
             g_ref, be_ref, o_ref) = refs
            base = e_ref[...]
        w1 = w1_ref[...]
        h = (jnp.dot(a_ref[...], w1[0:HID], preferred_element_type=jnp.float32)
             + jnp.dot(b_ref[...], w1[HID:2 * HID], preferred_element_type=jnp.float32)
             + jnp.dot(base, w1[2 * HID:3 * HID], preferred_element_type=jnp.float32)
             + b1_ref[...])
        h = jnp.maximum(h, 0.0)
        h = jnp.dot(h, w2_ref[...], preferred_element_type=jnp.float32) + b2_ref[...]
        o_ref[...] = base + _ln(h, g_ref[...], be_ref[...])

    row_spec = pl.BlockSpec((block, HID), lambda i: (i, 0))
    in_specs = [row_spec, row_spec, row_spec]
    args = [gxr, gxc, e]
    if has_cnt:
        in_specs += [pl.BlockSpec((block, 1), lambda i: (i, 0)),
                     pl.BlockSpec((block, 1), lambda i: (i, 0))]
        args += [cnt[0], cnt[1]]
    in_specs += [
        pl.BlockSpec((3 * HID, HID), lambda i: (0, 0)),
        pl.BlockSpec((1, HID), lambda i: (0, 0)),
        pl.BlockSpec((HID, HID), lambda i: (0, 0)),
        pl.BlockSpec((1, HID), lambda i: (0, 0)),
        pl.BlockSpec((1, HID), lambda i: (0, 0)),
        pl.BlockSpec((1, HID), lambda i: (0, 0)),
    ]
    args += [p['W1'], p['b1'].reshape(1, HID), p['W2'], p['b2'].reshape(1, HID),
             p['g'].reshape(1, HID), p['be'].reshape(1, HID)]
    return pl.pallas_call(
        body,
        grid=(n // block,),
        in_specs=in_specs,
        out_specs=row_spec,
        out_shape=jax.ShapeDtypeStruct((n, HID), jnp.float32),
    )(*args)


def _node_mlp_body(x_ref, a_ref, b_ref, w1_ref, b1_ref, w2_ref, b2_ref,
                   g_ref, be_ref):
    agg = a_ref[...] + b_ref[...]
    w1 = w1_ref[...]
    h = (jnp.dot(x_ref[...], w1[0:HID], preferred_element_type=jnp.float32)
         + jnp.dot(agg, w1[HID:2 * HID], preferred_element_type=jnp.float32)
         + b1_ref[...])
    h = jnp.maximum(h, 0.0)
    h = jnp.dot(h, w2_ref[...], preferred_element_type=jnp.float32) + b2_ref[...]
    return x_ref[...] + _ln(h, g_ref[...], be_ref[...])


def _node_specs(block):
    row_spec = pl.BlockSpec((block, HID), lambda i: (i, 0))
    return [
        row_spec, row_spec, row_spec,
        pl.BlockSpec((2 * HID, HID), lambda i: (0, 0)),
        pl.BlockSpec((1, HID), lambda i: (0, 0)),
        pl.BlockSpec((HID, HID), lambda i: (0, 0)),
        pl.BlockSpec((1, HID), lambda i: (0, 0)),
        pl.BlockSpec((1, HID), lambda i: (0, 0)),
        pl.BlockSpec((1, HID), lambda i: (0, 0)),
    ]


def _node_args(x, agg_a, agg_b, p):
    return [x, agg_a, agg_b, p['W1'], p['b1'].reshape(1, HID), p['W2'],
            p['b2'].reshape(1, HID), p['g'].reshape(1, HID), p['be'].reshape(1, HID)]


def _node_mlp(x, agg_a, agg_b, p, block=2048):
    n = x.shape[0]

    def body(x_ref, a_ref, b_ref, w1_ref, b1_ref, w2_ref, b2_ref, g_ref,
             be_ref, o_ref):
        o_ref[...] = _node_mlp_body(x_ref, a_ref, b_ref, w1_ref, b1_ref,
                                    w2_ref, b2_ref, g_ref, be_ref)

    return pl.pallas_call(
        body,
        grid=(n // block,),
        in_specs=_node_specs(block),
        out_specs=pl.BlockSpec((block, HID), lambda i: (i, 0)),
        out_shape=jax.ShapeDtypeStruct((n, HID), jnp.float32),
    )(*_node_args(x, agg_a, agg_b, p))


def _node_mlp_unpool(x, agg_a, agg_b, p, skip3, block=1024):
    n = x.shape[0]

    def body(x_ref, a_ref, b_ref, w1_ref, b1_ref, w2_ref, b2_ref, g_ref,
             be_ref, s_ref, o_ref):
        x_new = _node_mlp_body(x_ref, a_ref, b_ref, w1_ref, b1_ref, w2_ref,
                               b2_ref, g_ref, be_ref)
        o_ref[...] = x_new[:, None, :] + s_ref[...]

    return pl.pallas_call(
        body,
        grid=(n // block,),
        in_specs=_node_specs(block) + [pl.BlockSpec((block, 2, HID), lambda i: (i, 0, 0))],
        out_specs=pl.BlockSpec((block, 2, HID), lambda i: (i, 0, 0)),
        out_shape=jax.ShapeDtypeStruct((n, 2, HID), jnp.float32),
    )(*_node_args(x, agg_a, agg_b, p), skip3)


def _node_mlp_dec(x, agg_a, agg_b, p, pdec, block=2048):
    n = x.shape[0]

    def body(x_ref, a_ref, b_ref, w1_ref, b1_ref, w2_ref, b2_ref, g_ref,
             be_ref, d1_ref, db1_ref, d2_ref, db2_ref, o_ref):
        x_new = _node_mlp_body(x_ref, a_ref, b_ref, w1_ref, b1_ref, w2_ref,
                               b2_ref, g_ref, be_ref)
        h = jnp.dot(x_new, d1_ref[...], preferred_element_type=jnp.float32)
        h = jnp.maximum(h + db1_ref[...], 0.0)
        o_ref[...] = jnp.dot(h, d2_ref[...], preferred_element_type=jnp.float32) + db2_ref[...]

    return pl.pallas_call(
        body,
        grid=(n // block,),
        in_specs=_node_specs(block) + [
            pl.BlockSpec((HID, HID), lambda i: (0, 0)),
            pl.BlockSpec((1, HID), lambda i: (0, 0)),
            pl.BlockSpec((HID, 3), lambda i: (0, 0)),
            pl.BlockSpec((1, 3), lambda i: (0, 0)),
        ],
        out_specs=pl.BlockSpec((block, 3), lambda i: (i, 0)),
        out_shape=jax.ShapeDtypeStruct((n, 3), jnp.float32),
    )(*_node_args(x, agg_a, agg_b, p), pdec['W1'], pdec['b1'].reshape(1, HID),
      pdec['W2'], pdec['b2'].reshape(1, 3))


def _pool(x3, block=1024):
    n = x3.shape[0]

    def body(x_ref, o_ref):
        o_ref[...] = 0.5 * (x_ref[:, 0, :] + x_ref[:, 1, :])

    return pl.pallas_call(
        body,
        grid=(n // block,),
        in_specs=[pl.BlockSpec((block, 2, HID), lambda i: (i, 0, 0))],
        out_specs=pl.BlockSpec((block, HID), lambda i: (i, 0)),
        out_shape=jax.ShapeDtypeStruct((n, HID), jnp.float32),
    )(x3)


@functools.cache
def _mesh():
    return plsc.VectorSubcoreMesh(core_axis_name="c", subcore_axis_name="s")


@functools.cache
def _make_gather_pair():
    @functools.partial(
        pl.kernel, mesh=_mesh(),
        out_type=[jax.ShapeDtypeStruct((EP, HID), jnp.float32),
                  jax.ShapeDtypeStruct((EP, HID), jnp.float32)],
        scratch_types=[pltpu.VMEM((EPW // 128, 128), jnp.int32),
                       pltpu.VMEM((1024, HID), jnp.float32),
                       pltpu.SemaphoreType.DMA],
        compiler_params=pltpu.CompilerParams(use_tc_tiling_on_sc=False),
    )
    def k(x_hbm, idxa_hbm, idxb_hbm, outa_hbm, outb_hbm, idx_v, rows_v, sem):
        wid = lax.axis_index("s") * 2 + lax.axis_index("c")
        base = wid * EPW
        for idx_hbm, out_hbm in ((idxa_hbm, outa_hbm), (idxb_hbm, outb_hbm)):
            pltpu.sync_copy(idx_hbm.at[wid], idx_v)

            def g_body(g, carry):
                cps = [pltpu.async_copy(x_hbm.at[idx_v.at[g * 8 + j]],
                                        rows_v.at[pl.ds(j * 128, 128)], sem)
                       for j in range(8)]
                for cp in cps:
                    cp.wait()
                pltpu.sync_copy(rows_v, out_hbm.at[pl.ds(base + g * 1024, 1024)])
                return carry

            lax.fori_loop(0, EPW // 1024, g_body, 0)

    return k


def _sc_gather_pair(x, idxa, idxb):
    return _make_gather_pair()(x, idxa, idxb)


@functools.cache
def _make_scatter(acc_rows):
    rpt = acc_rows // 16

    @functools.partial(
        pl.kernel, mesh=_mesh(),
        out_type=jax.ShapeDtypeStruct((2, acc_rows, HID), jnp.float32),
        scratch_types=[pltpu.VMEM_SHARED((acc_rows, HID), jnp.float32),
                       pltpu.VMEM((EPW // 128, 128), jnp.int32),
                       pltpu.VMEM((512, HID), jnp.float32)],
        compiler_params=pltpu.CompilerParams(use_tc_tiling_on_sc=False),
    )
    def k(vals_hbm, idx_hbm, zeros_hbm, out_hbm, acc_s, idx_v, ebuf):
        cid = lax.axis_index("c")
        sid = lax.axis_index("s")
        wid = sid * 2 + cid
        base = wid * EPW
        pltpu.sync_copy(zeros_hbm.at[pl.ds(sid * rpt, rpt)],
                        acc_s.at[pl.ds(sid * rpt, rpt)])
        plsc.subcore_barrier()
        pltpu.sync_copy(idx_hbm.at[wid], idx_v)

        def g_body(g, carry):
            pltpu.sync_copy(vals_hbm.at[pl.ds(base + g * 512, 512)], ebuf)
            for j in range(4):
                pltpu.sync_copy(ebuf.at[pl.ds(j * 128, 128)],
                                acc_s.at[idx_v.at[g * 4 + j]], add=True)
            return carry

        lax.fori_loop(0, EPW // 512, g_body, 0)
        plsc.subcore_barrier()
        pltpu.sync_copy(acc_s.at[pl.ds(sid * rpt, rpt)],
                        out_hbm.at[cid, pl.ds(sid * rpt, rpt)])

    return k


def _sc_scatter_fine(vals, idx, zeros):
    return _make_scatter(ACC_F)(vals, idx, zeros)


def _sc_scatter_coarse(vals, idx, zeros):
    return _make_scatter(ACC_C)(vals, idx, zeros)


NKEY = NC * NC + 1
NCHUNK = EPW // 128


@functools.cache
def _make_dedup_scatter():
    @functools.partial(
        pl.kernel, mesh=_mesh(),
        out_type=jax.ShapeDtypeStruct((NKEY,), jnp.int32),
        scratch_types=[pltpu.VMEM((NCHUNK, 128), jnp.int32),
                       pltpu.VMEM((EPW,), jnp.int32)],
        compiler_params=pltpu.CompilerParams(use_tc_tiling_on_sc=False),
    )
    def k(keys_hbm, ids_hbm, table_hbm, keys_v, ids_v):
        wid = lax.axis_index("s") * 2 + lax.axis_index("c")
        pltpu.sync_copy(keys_hbm.at[wid], keys_v)
        pltpu.sync_copy(ids_hbm.at[wid], ids_v)

        def body(j, c):
            pltpu.sync_copy(ids_v.at[pl.ds(j * 128, 128)],
                            table_hbm.at[keys_v.at[j]])
            return c

        lax.fori_loop(0, NCHUNK, body, 0)

    return k


def _sc_dedup_scatter(keys3, ids3):
    return _make_dedup_scatter()(keys3, ids3)


@functools.cache
def _make_dedup_gather():
    @functools.partial(
        pl.kernel, mesh=_mesh(),
        out_type=[jax.ShapeDtypeStruct((NW, EPW), jnp.int32),
                  jax.ShapeDtypeStruct((NW, NCHUNK, 128), jnp.int32),
                  jax.ShapeDtypeStruct((2, EP), jnp.float32)],
        scratch_types=[pltpu.VMEM((NCHUNK, 128), jnp.int32),
                       pltpu.VMEM((NCHUNK, 128), jnp.int32),
                       pltpu.VMEM((NCHUNK, 128), jnp.int32),
                       pltpu.VMEM((NCHUNK, 128), jnp.int32),
                       pltpu.VMEM((128,), jnp.int32),
                       pltpu.VMEM((128,), jnp.float32),
                       pltpu.VMEM_SHARED((EP,), jnp.float32)],
        compiler_params=pltpu.CompilerParams(use_tc_tiling_on_sc=False),
    )
    def k(keys_hbm, ccol_hbm, table_hbm, zeros_hbm, rep_hbm, ccols_hbm,
          cnt_hbm, keys_v, ccol_v, repidx_v, ccs_v, chunk_v, ones_v, acc_s):
        cid = lax.axis_index("c")
        sid = lax.axis_index("s")
        wid = sid * 2 + cid
        rpt = EP // 16
        pltpu.sync_copy(zeros_hbm.at[pl.ds(sid * rpt, rpt)],
                        acc_s.at[pl.ds(sid * rpt, rpt)])
        for kk in range(8):
            ones_v[pl.ds(kk * 16, 16)] = jnp.full((16,), 1.0, jnp.float32)
        pltpu.sync_copy(keys_hbm.at[wid], keys_v)
        pltpu.sync_copy(ccol_hbm.at[wid], ccol_v)
        plsc.subcore_barrier()
        base = wid * EPW
        lane = lax.iota(jnp.int32, 16)
        for j in range(NCHUNK):
            pltpu.sync_copy(table_hbm.at[keys_v.at[j]], chunk_v)
            for kk in range(8):
                rep = chunk_v[pl.ds(kk * 16, 16)]
                ids = base + j * 128 + kk * 16 + lane
                m = (rep == ids) & (ids < E)
                repidx_v[j, pl.ds(kk * 16, 16)] = rep
                ccs_v[j, pl.ds(kk * 16, 16)] = jnp.where(
                    m, ccol_v[j, pl.ds(kk * 16, 16)], NCP)
            pltpu.sync_copy(chunk_v, rep_hbm.at[wid, pl.ds(j * 128, 128)])
            pltpu.sync_copy(ones_v, acc_s.at[repidx_v.at[j]], add=True)
        pltpu.sync_copy(ccs_v, ccols_hbm.at[wid])
        plsc.subcore_barrier()
        pltpu.sync_copy(acc_s.at[pl.ds(sid * rpt, rpt)],
                        cnt_hbm.at[cid, pl.ds(sid * rpt, rpt)])

    return k


def _sc_dedup_gather(keys3, ccol3, table, zeros_cnt):
    return _make_dedup_gather()(keys3, ccol3, table, zeros_cnt)


CE_HALF = EP // 2
CE_ACC = CE_HALF + 128
CE_RPT = CE_ACC // 16
EPT = EP // 16


@functools.cache
def _make_ce_scatter():
    @functools.partial(
        pl.kernel, mesh=_mesh(),
        out_type=jax.ShapeDtypeStruct((EP, HID), jnp.float32),
        scratch_types=[pltpu.VMEM((EPT // 128, 128), jnp.int32),
                       pltpu.VMEM((512, 16), jnp.float32),
                       pltpu.VMEM_SHARED((CE_ACC, 16), jnp.float32)],
        compiler_params=pltpu.CompilerParams(use_tc_tiling_on_sc=False),
    )
    def k(vals_hbm, rep_hbm, zeros_hbm, out_hbm, idx_v, ebuf, acc_s):
        cid = lax.axis_index("c")
        sid = lax.axis_index("s")
        tbase = sid * EPT
        pltpu.sync_copy(rep_hbm.at[2 * sid], idx_v.at[pl.ds(0, NCHUNK)])
        pltpu.sync_copy(rep_hbm.at[2 * sid + 1],
                        idx_v.at[pl.ds(NCHUNK, NCHUNK)])
        lo = cid * CE_HALF

        def loc_body(j, c):
            for kk in range(8):
                v = idx_v[j, pl.ds(kk * 16, 16)]
                il = v - lo
                m = (il >= 0) & (il < CE_HALF)
                idx_v[j, pl.ds(kk * 16, 16)] = jnp.where(m, il, CE_HALF)
            return c

        lax.fori_loop(0, EPT // 128, loc_body, 0)
        for p in range(4):
            pltpu.sync_copy(zeros_hbm.at[pl.ds(sid * CE_RPT, CE_RPT)],
                            acc_s.at[pl.ds(sid * CE_RPT, CE_RPT)])
            plsc.subcore_barrier()

            def body(gq, c):
                pltpu.sync_copy(
                    vals_hbm.at[pl.ds(tbase + gq * 512, 512), pl.ds(p * 16, 16)],
                    ebuf)
                for j4 in range(4):
                    pltpu.sync_copy(ebuf.at[pl.ds(j4 * 128, 128)],
                                    acc_s.at[idx_v.at[gq * 4 + j4]], add=True)
                return c

            lax.fori_loop(0, EPT // 512, body, 0)
            plsc.subcore_barrier()
            pltpu.sync_copy(
                acc_s.at[pl.ds(sid * (CE_HALF // 16), CE_HALF // 16)],
                out_hbm.at[pl.ds(cid * CE_HALF + sid * (CE_HALF // 16),
                                 CE_HALF // 16), pl.ds(p * 16, 16)])
            plsc.subcore_barrier()

    return k


def _sc_ce_scatter(vals, rep3, zeros):
    return _make_ce_scatter()(vals, rep3, zeros)


def _r3(a):
    return a.astype(jnp.int32).reshape(NW, EPW // 128, 128)


def kernel(node_attr, edge_attr, edge_index, params):
    f32 = jnp.float32
    p = params
    eid = jnp.arange(EP, dtype=jnp.int32)
    real = eid < E

    nap = jnp.pad(node_attr.astype(f32), ((0, NP - N_FINE), (0, 0)))
    eap = jnp.pad(edge_attr.astype(f32), ((0, EP - E), (0, 0)))
    rowp = jnp.pad(edge_index[0].astype(jnp.int32), (0, EP - E))
    colp = jnp.pad(edge_index[1].astype(jnp.int32), (0, EP - E))
    col_s = jnp.where(real, colp, NP)
    crow = rowp // 2
    ccol = colp // 2

    keys = jnp.where(real, crow * NC + ccol, NC * NC)
    table = _sc_dedup_scatter(_r3(keys), eid.reshape(NW, EPW))
    rep_flat, idx_ccol_s, cnts = _sc_dedup_gather(
        _r3(keys), _r3(ccol), table, jnp.zeros((EP,), f32))
    rep3 = rep_flat.reshape(NW, NCHUNK, 128)
    cnt2 = (cnts[0].reshape(EP, 1), cnts[1].reshape(EP, 1))

    idx_row = _r3(rowp)
    idx_col = _r3(colp)
    idx_col_s = _r3(col_s)
    idx_crow = _r3(crow)
    idx_ccol = _r3(ccol)
    zeros_f = jnp.zeros((ACC_F, HID), f32)
    zeros_c = jnp.zeros((ACC_C, HID), f32)
    zeros_ce = jnp.zeros((CE_ACC, 16), f32)

    x0 = _enc(nap, p['node_enc'], block=2048)
    e0 = _enc(eap, p['edge_enc'], block=2048)

    lp = p['down'][0]
    gxr, gxc = _sc_gather_pair(x0, idx_row, idx_col)
    e1 = _edge_mlp(gxr, gxc, e0, lp['edge'])
    parts = _sc_scatter_fine(e1, idx_col_s, zeros_f)
    x1 = _node_mlp(x0, parts[0, :NP], parts[1, :NP], lp['node'], block=2048)

    cx = _pool(x1.reshape(NCP, 2, HID))
    ce_sums = _sc_ce_scatter(e1, rep3, zeros_ce)

    lp = p['bottleneck'][0]
    cgr, cgc = _sc_gather_pair(cx, idx_crow, idx_ccol)
    ec = _edge_mlp(cgr, cgc, ce_sums, lp['edge'], cnt=cnt2)
    cparts = _sc_scatter_coarse(ec, idx_ccol_s, zeros_c)
    cx = _node_mlp(cx, cparts[0, :NCP], cparts[1, :NCP], lp['node'], block=1024)

    lp = p['bottleneck'][1]
    cgr, cgc = _sc_gather_pair(cx, idx_crow, idx_ccol)
    ec = _edge_mlp(cgr, cgc, ec, lp['edge'])
    cparts = _sc_scatter_coarse(ec, idx_ccol_s, zeros_c)
    xup3 = _node_mlp_unpool(cx, cparts[0, :NCP], cparts[1, :NCP], lp['node'],
                            x1.reshape(NCP, 2, HID), block=1024)
    xup = xup3.reshape(NP, HID)

    lp = p['up'][0]
    gxr, gxc = _sc_gather_pair(xup, idx_row, idx_col)
    eu = _edge_mlp(gxr, gxc, e1, lp['edge'])
    parts = _sc_scatter_fine(eu, idx_col_s, zeros_f)
    out = _node_mlp_dec(xup, parts[0, :NP], parts[1, :NP], lp['node'],
                        p['dec'], block=2048)
    return out[:N_FINE]


# speedup vs baseline: 2.2063x; 1.0305x over previous
import functools

import jax
import jax.numpy as jnp
from jax import lax
from jax.experimental import pallas as pl
from jax.experimental.pallas import tpu as pltpu
from jax.experimental.pallas import tpu_sc as plsc

HID = 64
N_FINE = 10000
NP = 10240
E = 160000
EP = 163840
NC = 5000
NCP = 5120
ACC_F = 10256
ACC_C = 5136
NW = 32
EPW = EP // NW
LN_EPS = 1e-5


def _ln(h, g, be):
    mu = jnp.mean(h, axis=-1, keepdims=True)
    var = jnp.mean((h - mu) ** 2, axis=-1, keepdims=True)
    return (h - mu) * lax.rsqrt(var + LN_EPS) * g + be


def _enc(x, p, block):
    n, din = x.shape

    def body(x_ref, w1_ref, b1_ref, w2_ref, b2_ref, g_ref, be_ref, o_ref):
        h = jnp.dot(x_ref[...], w1_ref[...], preferred_element_type=jnp.float32)
        h = jnp.maximum(h + b1_ref[...], 0.0)
        h = jnp.dot(h, w2_ref[...], preferred_element_type=jnp.float32) + b2_ref[...]
        o_ref[...] = _ln(h, g_ref[...], be_ref[...])

    return pl.pallas_call(
        body,
        grid=(n // block,),
        in_specs=[
            pl.BlockSpec((block, din), lambda i: (i, 0)),
            pl.BlockSpec((din, HID), lambda i: (0, 0)),
            pl.BlockSpec((1, HID), lambda i: (0, 0)),
            pl.BlockSpec((HID, HID), lambda i: (0, 0)),
            pl.BlockSpec((1, HID), lambda i: (0, 0)),
            pl.BlockSpec((1, HID), lambda i: (0, 0)),
            pl.BlockSpec((1, HID), lambda i: (0, 0)),
        ],
        out_specs=pl.BlockSpec((block, HID), lambda i: (i, 0)),
        out_shape=jax.ShapeDtypeStruct((n, HID), jnp.float32),
    )(x, p['W1'], p['b1'].reshape(1, HID), p['W2'], p['b2'].reshape(1, HID),
      p['g'].reshape(1, HID), p['be'].reshape(1, HID))


def _edge_mlp(gxr, gxc, e, p, cnt=None, block=2048):
    n = gxr.shape[0]
    has_cnt = cnt is not None

    def body(*refs):
        if has_cnt:
            (a_ref, b_ref, e_ref, ca_ref, cb_ref, w1_ref, b1_ref, w2_ref,
             b2_ref, g_ref, be_ref, o_ref) = refs
            base = e_ref[...] * (1.0 / jnp.maximum(ca_ref[...] + cb_ref[...],
                                                   1.0))
        else:
            (a_ref, b_ref, e_ref, w1_ref, b1_ref, w2_ref, b2_ref,
             g_ref, be_ref, o_ref) = refs
            base = e_ref[...]
        w1 = w1_ref[...]
        h = (jnp.dot(a_ref[...], w1[0:HID], preferred_element_type=jnp.float32)
             + jnp.dot(b_ref[...], w1[HID:2 * HID], preferred_element_type=jnp.float32)
             + jnp.dot(base, w1[2 * HID:3 * HID], preferred_element_type=jnp.float32)
             + b1_ref[...])
        h = jnp.maximum(h, 0.0)
        h = jnp.dot(h, w2_ref[...], preferred_element_type=jnp.float32) + b2_ref[...]
        o_ref[...] = base + _ln(h, g_ref[...], be_ref[...])

    row_spec = pl.BlockSpec((block, HID), lambda i: (i, 0))
    in_specs = [row_spec, row_spec, row_spec]
    args = [gxr, gxc, e]
    if has_cnt:
        in_specs += [pl.BlockSpec((block, 1), lambda i: (i, 0)),
                     pl.BlockSpec((block, 1), lambda i: (i, 0))]
        args += [cnt[0], cnt[1]]
    in_specs += [
        pl.BlockSpec((3 * HID, HID), lambda i: (0, 0)),
        pl.BlockSpec((1, HID), lambda i: (0, 0)),
        pl.BlockSpec((HID, HID), lambda i: (0, 0)),
        pl.BlockSpec((1, HID), lambda i: (0, 0)),
        pl.BlockSpec((1, HID), lambda i: (0, 0)),
        pl.BlockSpec((1, HID), lambda i: (0, 0)),
    ]
    args += [p['W1'], p['b1'].reshape(1, HID), p['W2'], p['b2'].reshape(1, HID),
             p['g'].reshape(1, HID), p['be'].reshape(1, HID)]
    return pl.pallas_call(
        body,
        grid=(n // block,),
        in_specs=in_specs,
        out_specs=row_spec,
        out_shape=jax.ShapeDtypeStruct((n, HID), jnp.float32),
    )(*args)


def _node_mlp_body(x_ref, a_ref, b_ref, w1_ref, b1_ref, w2_ref, b2_ref,
                   g_ref, be_ref):
    agg = a_ref[...] + b_ref[...]
    w1 = w1_ref[...]
    h = (jnp.dot(x_ref[...], w1[0:HID], preferred_element_type=jnp.float32)
         + jnp.dot(agg, w1[HID:2 * HID], preferred_element_type=jnp.float32)
         + b1_ref[...])
    h = jnp.maximum(h, 0.0)
    h = jnp.dot(h, w2_ref[...], preferred_element_type=jnp.float32) + b2_ref[...]
    return x_ref[...] + _ln(h, g_ref[...], be_ref[...])


def _node_specs(block):
    row_spec = pl.BlockSpec((block, HID), lambda i: (i, 0))
    return [
        row_spec, row_spec, row_spec,
        pl.BlockSpec((2 * HID, HID), lambda i: (0, 0)),
        pl.BlockSpec((1, HID), lambda i: (0, 0)),
        pl.BlockSpec((HID, HID), lambda i: (0, 0)),
        pl.BlockSpec((1, HID), lambda i: (0, 0)),
        pl.BlockSpec((1, HID), lambda i: (0, 0)),
        pl.BlockSpec((1, HID), lambda i: (0, 0)),
    ]


def _node_args(x, agg_a, agg_b, p):
    return [x, agg_a, agg_b, p['W1'], p['b1'].reshape(1, HID), p['W2'],
            p['b2'].reshape(1, HID), p['g'].reshape(1, HID), p['be'].reshape(1, HID)]


def _node_mlp(x, agg_a, agg_b, p, block=2048):
    n = x.shape[0]

    def body(x_ref, a_ref, b_ref, w1_ref, b1_ref, w2_ref, b2_ref, g_ref,
             be_ref, o_ref):
        o_ref[...] = _node_mlp_body(x_ref, a_ref, b_ref, w1_ref, b1_ref,
                                    w2_ref, b2_ref, g_ref, be_ref)

    return pl.pallas_call(
        body,
        grid=(n // block,),
        in_specs=_node_specs(block),
        out_specs=pl.BlockSpec((block, HID), lambda i: (i, 0)),
        out_shape=jax.ShapeDtypeStruct((n, HID), jnp.float32),
    )(*_node_args(x, agg_a, agg_b, p))


def _node_mlp_unpool(x, agg_a, agg_b, p, skip3, block=1024):
    n = x.shape[0]

    def body(x_ref, a_ref, b_ref, w1_ref, b1_ref, w2_ref, b2_ref, g_ref,
             be_ref, s_ref, o_ref):
        x_new = _node_mlp_body(x_ref, a_ref, b_ref, w1_ref, b1_ref, w2_ref,
                               b2_ref, g_ref, be_ref)
        o_ref[...] = x_new[:, None, :] + s_ref[...]

    return pl.pallas_call(
        body,
        grid=(n // block,),
        in_specs=_node_specs(block) + [pl.BlockSpec((block, 2, HID), lambda i: (i, 0, 0))],
        out_specs=pl.BlockSpec((block, 2, HID), lambda i: (i, 0, 0)),
        out_shape=jax.ShapeDtypeStruct((n, 2, HID), jnp.float32),
    )(*_node_args(x, agg_a, agg_b, p), skip3)


def _node_mlp_dec(x, agg_a, agg_b, p, pdec, block=2048):
    n = x.shape[0]

    def body(x_ref, a_ref, b_ref, w1_ref, b1_ref, w2_ref, b2_ref, g_ref,
             be_ref, d1_ref, db1_ref, d2_ref, db2_ref, o_ref):
        x_new = _node_mlp_body(x_ref, a_ref, b_ref, w1_ref, b1_ref, w2_ref,
                               b2_ref, g_ref, be_ref)
        h = jnp.dot(x_new, d1_ref[...], preferred_element_type=jnp.float32)
        h = jnp.maximum(h + db1_ref[...], 0.0)
        o_ref[...] = jnp.dot(h, d2_ref[...], preferred_element_type=jnp.float32) + db2_ref[...]

    return pl.pallas_call(
        body,
        grid=(n // block,),
        in_specs=_node_specs(block) + [
            pl.BlockSpec((HID, HID), lambda i: (0, 0)),
            pl.BlockSpec((1, HID), lambda i: (0, 0)),
            pl.BlockSpec((HID, 3), lambda i: (0, 0)),
            pl.BlockSpec((1, 3), lambda i: (0, 0)),
        ],
        out_specs=pl.BlockSpec((block, 3), lambda i: (i, 0)),
        out_shape=jax.ShapeDtypeStruct((n, 3), jnp.float32),
    )(*_node_args(x, agg_a, agg_b, p), pdec['W1'], pdec['b1'].reshape(1, HID),
      pdec['W2'], pdec['b2'].reshape(1, 3))


def _pool(x3, block=1024):
    n = x3.shape[0]

    def body(x_ref, o_ref):
        o_ref[...] = 0.5 * (x_ref[:, 0, :] + x_ref[:, 1, :])

    return pl.pallas_call(
        body,
        grid=(n // block,),
        in_specs=[pl.BlockSpec((block, 2, HID), lambda i: (i, 0, 0))],
        out_specs=pl.BlockSpec((block, HID), lambda i: (i, 0)),
        out_shape=jax.ShapeDtypeStruct((n, HID), jnp.float32),
    )(x3)


@functools.cache
def _mesh():
    return plsc.VectorSubcoreMesh(core_axis_name="c", subcore_axis_name="s")


GG = 640
NGG = EPW // GG


@functools.cache
def _make_gather_pair():
    @functools.partial(
        pl.kernel, mesh=_mesh(), name="sc_gather_pair",
        out_type=[jax.ShapeDtypeStruct((EP, HID), jnp.float32),
                  jax.ShapeDtypeStruct((EP, HID), jnp.float32)],
        scratch_types=[pltpu.VMEM((EPW,), jnp.int32),
                       pltpu.VMEM((2 * GG, HID), jnp.float32),
                       pltpu.SemaphoreType.DMA],
        compiler_params=pltpu.CompilerParams(use_tc_tiling_on_sc=False),
    )
    def k(x_hbm, idxa_hbm, idxb_hbm, outa_hbm, outb_hbm, idx_v, rows_v, sem):
        wid = lax.axis_index("s") * 2 + lax.axis_index("c")
        base = wid * EPW

        def fire(g, b):
            pltpu.async_copy(x_hbm.at[idx_v.at[pl.ds(g * GG, GG)]],
                             rows_v.at[pl.ds(b * GG, GG)], sem)

        def drain(b):
            pltpu.make_async_copy(x_hbm.at[idx_v.at[pl.ds(0, GG)]],
                                  rows_v.at[pl.ds(b * GG, GG)], sem).wait()

        for idx_hbm, out_hbm in ((idxa_hbm, outa_hbm), (idxb_hbm, outb_hbm)):
            pltpu.sync_copy(idx_hbm.at[wid], idx_v)
            fire(0, 0)

            def g_body(g, carry):
                b = lax.rem(g, 2)
                fire(g, b)
                drain(1 - b)
                pltpu.sync_copy(rows_v.at[pl.ds((1 - b) * GG, GG)],
                                out_hbm.at[pl.ds(base + (g - 1) * GG, GG)])
                return carry

            lax.fori_loop(1, NGG, g_body, 0)
            b_last = (NGG - 1) % 2
            drain(b_last)
            pltpu.sync_copy(rows_v.at[pl.ds(b_last * GG, GG)],
                            out_hbm.at[pl.ds(base + (NGG - 1) * GG, GG)])

    return k


def _sc_gather_pair(x, idxa, idxb):
    return _make_gather_pair()(x, idxa, idxb)


@functools.cache
def _make_scatter(acc_rows):
    rpt = acc_rows // 16
    ngs = EPW // 512

    @functools.partial(
        pl.kernel, mesh=_mesh(), name=f"sc_scatter_{acc_rows}",
        out_type=jax.ShapeDtypeStruct((2, acc_rows, HID), jnp.float32),
        scratch_types=[pltpu.VMEM_SHARED((acc_rows, HID), jnp.float32),
                       pltpu.VMEM((EPW // 128, 128), jnp.int32),
                       pltpu.VMEM((1024, HID), jnp.float32),
                       pltpu.SemaphoreType.DMA,
                       pltpu.SemaphoreType.DMA],
        compiler_params=pltpu.CompilerParams(use_tc_tiling_on_sc=False),
    )
    def k(vals_hbm, idx_hbm, zeros_hbm, out_hbm, acc_s, idx_v, ebuf,
          lsem, ssem):
        cid = lax.axis_index("c")
        sid = lax.axis_index("s")
        wid = sid * 2 + cid
        base = wid * EPW

        def fire_load(g, b):
            pltpu.async_copy(vals_hbm.at[pl.ds(base + g * 512, 512)],
                             ebuf.at[pl.ds(b * 512, 512)], lsem)

        def drain_load(b):
            pltpu.make_async_copy(vals_hbm.at[pl.ds(base, 512)],
                                  ebuf.at[pl.ds(b * 512, 512)], lsem).wait()

        def drain_adds(b):
            pltpu.make_async_copy(vals_hbm.at[pl.ds(base, 512)],
                                  ebuf.at[pl.ds(b * 512, 512)], ssem).wait()

        pltpu.sync_copy(idx_hbm.at[wid], idx_v)
        fire_load(0, 0)
        pltpu.sync_copy(zeros_hbm.at[pl.ds(sid * rpt, rpt)],
                        acc_s.at[pl.ds(sid * rpt, rpt)])
        plsc.subcore_barrier()

        def g_body(g, carry):
            b = lax.rem(g, 2)
            fire_load(g + 1, 1 - b)
            drain_load(b)
            for j in range(4):
                pltpu.async_copy(ebuf.at[pl.ds(b * 512 + j * 128, 128)],
                                 acc_s.at[idx_v.at[g * 4 + j]], ssem,
                                 add=True)
            drain_adds(b)
            return carry

        lax.fori_loop(0, ngs - 1, g_body, 0)
        b = (ngs - 1) % 2
        drain_load(b)
        for j in range(4):
            pltpu.async_copy(ebuf.at[pl.ds(b * 512 + j * 128, 128)],
                             acc_s.at[idx_v.at[(ngs - 1) * 4 + j]], ssem,
                             add=True)
        drain_adds(b)
        plsc.subcore_barrier()
        pltpu.sync_copy(acc_s.at[pl.ds(sid * rpt, rpt)],
                        out_hbm.at[cid, pl.ds(sid * rpt, rpt)])

    return k


def _sc_scatter_fine(vals, idx, zeros):
    return _make_scatter(ACC_F)(vals, idx, zeros)


def _sc_scatter_coarse(vals, idx, zeros):
    return _make_scatter(ACC_C)(vals, idx, zeros)


NKEY = NC * NC + 1
NCHUNK = EPW // 128


@functools.cache
def _make_dedup_scatter():
    @functools.partial(
        pl.kernel, mesh=_mesh(), name="sc_dedup_scatter",
        out_type=jax.ShapeDtypeStruct((NKEY,), jnp.int32),
        scratch_types=[pltpu.VMEM((NCHUNK, 128), jnp.int32),
                       pltpu.VMEM((EPW,), jnp.int32),
                       pltpu.SemaphoreType.DMA],
        compiler_params=pltpu.CompilerParams(use_tc_tiling_on_sc=False),
    )
    def k(keys_hbm, ids_hbm, table_hbm, keys_v, ids_v, sem):
        wid = lax.axis_index("s") * 2 + lax.axis_index("c")
        pltpu.sync_copy(keys_hbm.at[wid], keys_v)
        pltpu.sync_copy(ids_hbm.at[wid], ids_v)

        def body(g, c):
            for j in range(4):
                pltpu.async_copy(ids_v.at[pl.ds((g * 4 + j) * 128, 128)],
                                 table_hbm.at[keys_v.at[g * 4 + j]], sem)
            for j in range(4):
                pltpu.make_async_copy(ids_v.at[pl.ds(j * 128, 128)],
                                      table_hbm.at[keys_v.at[g * 4 + j]],
                                      sem).wait()
            return c

        lax.fori_loop(0, NCHUNK // 4, body, 0)

    return k


def _sc_dedup_scatter(keys3, ids3):
    return _make_dedup_scatter()(keys3, ids3)


@functools.cache
def _make_dedup_gather():
    @functools.partial(
        pl.kernel, mesh=_mesh(),
        out_type=[jax.ShapeDtypeStruct((NW, EPW), jnp.int32),
                  jax.ShapeDtypeStruct((NW, NCHUNK, 128), jnp.int32),
                  jax.ShapeDtypeStruct((2, EP), jnp.float32)],
        scratch_types=[pltpu.VMEM((EPW,), jnp.int32),
                       pltpu.VMEM((NCHUNK, 128), jnp.int32),
                       pltpu.VMEM((NCHUNK, 128), jnp.int32),
                       pltpu.VMEM((NCHUNK, 128), jnp.int32),
                       pltpu.VMEM((EPW,), jnp.int32),
                       pltpu.VMEM((128,), jnp.float32),
                       pltpu.VMEM_SHARED((EP,), jnp.float32),
                       pltpu.SemaphoreType.DMA,
                       pltpu.SemaphoreType.DMA],
        compiler_params=pltpu.CompilerParams(use_tc_tiling_on_sc=False),
    )
    def k(keys_hbm, ccol_hbm, table_hbm, zeros_hbm, rep_hbm, ccols_hbm,
          cnt_hbm, keys_v, ccol_v, repidx_v, ccs_v, rep_v, ones_v, acc_s,
          gsem, ssem):
        cid = lax.axis_index("c")
        sid = lax.axis_index("s")
        wid = sid * 2 + cid
        rpt = EP // 16
        pltpu.sync_copy(keys_hbm.at[wid], keys_v)
        pltpu.sync_copy(ccol_hbm.at[wid], ccol_v)
        for g in range(EPW // 1024):
            pltpu.async_copy(table_hbm.at[keys_v.at[pl.ds(g * 1024, 1024)]],
                             rep_v.at[pl.ds(g * 1024, 1024)], gsem)
        pltpu.sync_copy(zeros_hbm.at[pl.ds(sid * rpt, rpt)],
                        acc_s.at[pl.ds(sid * rpt, rpt)])
        for kk in range(8):
            ones_v[pl.ds(kk * 16, 16)] = jnp.full((16,), 1.0, jnp.float32)
        for g in range(EPW // 1024):
            pltpu.make_async_copy(table_hbm.at[keys_v.at[pl.ds(0, 1024)]],
                                  rep_v.at[pl.ds(g * 1024, 1024)], gsem).wait()
        plsc.subcore_barrier()
        base = wid * EPW
        lane = lax.iota(jnp.int32, 16)
        for j in range(NCHUNK):
            for kk in range(8):
                rep = rep_v[pl.ds(j * 128 + kk * 16, 16)]
                ids = base + j * 128 + kk * 16 + lane
                m = (rep == ids) & (ids < E)
                repidx_v[j, pl.ds(kk * 16, 16)] = rep
                ccs_v[j, pl.ds(kk * 16, 16)] = jnp.where(
                    m, ccol_v[j, pl.ds(kk * 16, 16)], NCP)
        pltpu.sync_copy(rep_v, rep_hbm.at[wid])
        pltpu.sync_copy(ccs_v, ccols_hbm.at[wid])

        def cnt_body(g, c):
            for j in range(4):
                pltpu.async_copy(ones_v, acc_s.at[repidx_v.at[g * 4 + j]],
                                 ssem, add=True)
            for j in range(4):
                pltpu.make_async_copy(ones_v,
                                      acc_s.at[repidx_v.at[g * 4 + j]],
                                      ssem).wait()
            return c

        lax.fori_loop(0, NCHUNK // 4, cnt_body, 0)
        plsc.subcore_barrier()
        pltpu.sync_copy(acc_s.at[pl.ds(sid * rpt, rpt)],
                        cnt_hbm.at[cid, pl.ds(sid * rpt, rpt)])

    return k


def _sc_dedup_gather(keys3, ccol3, table, zeros_cnt):
    return _make_dedup_gather()(keys3, ccol3, table, zeros_cnt)


CE_HALF = EP // 2
CE_ACC = CE_HALF + 128
CE_RPT = CE_ACC // 16
EPT = EP // 16


@functools.cache
def _make_ce_scatter():
    @functools.partial(
        pl.kernel, mesh=_mesh(), name="sc_ce_scatter",
        out_type=jax.ShapeDtypeStruct((EP, HID), jnp.float32),
        scratch_types=[pltpu.VMEM((EPT // 128, 128), jnp.int32),
                       pltpu.VMEM((1024, 16), jnp.float32),
                       pltpu.VMEM_SHARED((CE_ACC, 16), jnp.float32),
                       pltpu.SemaphoreType.DMA,
                       pltpu.SemaphoreType.DMA],
        compiler_params=pltpu.CompilerParams(use_tc_tiling_on_sc=False),
    )
    def k(vals_hbm, rep_hbm, zeros_hbm, out_hbm, idx_v, ebuf, acc_s,
          lsem, ssem):
        cid = lax.axis_index("c")
        sid = lax.axis_index("s")
        tbase = sid * EPT
        ngs = EPT // 512
        pltpu.sync_copy(rep_hbm.at[2 * sid], idx_v.at[pl.ds(0, NCHUNK)])
        pltpu.sync_copy(rep_hbm.at[2 * sid + 1],
                        idx_v.at[pl.ds(NCHUNK, NCHUNK)])
        lo = cid * CE_HALF

        def loc_body(j, c):
            for kk in range(8):
                v = idx_v[j, pl.ds(kk * 16, 16)]
                il = v - lo
                m = (il >= 0) & (il < CE_HALF)
                idx_v[j, pl.ds(kk * 16, 16)] = jnp.where(m, il, CE_HALF)
            return c

        lax.fori_loop(0, EPT // 128, loc_body, 0)
        for p in range(4):

            def fire_load(g, b):
                pltpu.async_copy(
                    vals_hbm.at[pl.ds(tbase + g * 512, 512),
                                pl.ds(p * 16, 16)],
                    ebuf.at[pl.ds(b * 512, 512)], lsem)

            def drain(b, sem):
                pltpu.make_async_copy(
                    vals_hbm.at[pl.ds(tbase, 512), pl.ds(p * 16, 16)],
                    ebuf.at[pl.ds(b * 512, 512)], sem).wait()

            def adds(g, b):
                for j4 in range(4):
                    pltpu.async_copy(ebuf.at[pl.ds(b * 512 + j4 * 128, 128)],
                                     acc_s.at[idx_v.at[g * 4 + j4]], ssem,
                                     add=True)
                drain(b, ssem)

            fire_load(0, 0)
            pltpu.sync_copy(zeros_hbm.at[pl.ds(sid * CE_RPT, CE_RPT)],
                            acc_s.at[pl.ds(sid * CE_RPT, CE_RPT)])
            plsc.subcore_barrier()

            def body(g, c):
                b = lax.rem(g, 2)
                fire_load(g + 1, 1 - b)
                drain(b, lsem)
                adds(g, b)
                return c

            lax.fori_loop(0, ngs - 1, body, 0)
            b = (ngs - 1) % 2
            drain(b, lsem)
            adds(ngs - 1, b)
            plsc.subcore_barrier()
            pltpu.sync_copy(
                acc_s.at[pl.ds(sid * (CE_HALF // 16), CE_HALF // 16)],
                out_hbm.at[pl.ds(cid * CE_HALF + sid * (CE_HALF // 16),
                                 CE_HALF // 16), pl.ds(p * 16, 16)])
            plsc.subcore_barrier()

    return k


def _sc_ce_scatter(vals, rep3, zeros):
    return _make_ce_scatter()(vals, rep3, zeros)


def _r3(a):
    return a.astype(jnp.int32).reshape(NW, EPW // 128, 128)


def _r2(a):
    return a.astype(jnp.int32).reshape(NW, EPW)


def kernel(node_attr, edge_attr, edge_index, params):
    f32 = jnp.float32
    p = params
    eid = jnp.arange(EP, dtype=jnp.int32)
    real = eid < E

    nap = jnp.pad(node_attr.astype(f32), ((0, NP - N_FINE), (0, 0)))
    eap = jnp.pad(edge_attr.astype(f32), ((0, EP - E), (0, 0)))
    rowp = jnp.pad(edge_index[0].astype(jnp.int32), (0, EP - E))
    colp = jnp.pad(edge_index[1].astype(jnp.int32), (0, EP - E))
    col_s = jnp.where(real, colp, NP)
    crow = rowp // 2
    ccol = colp // 2

    keys = jnp.where(real, crow * NC + ccol, NC * NC)
    table = _sc_dedup_scatter(_r3(keys), eid.reshape(NW, EPW))
    rep_flat, idx_ccol_s, cnts = _sc_dedup_gather(
        _r2(keys), _r3(ccol), table, jnp.zeros((EP,), f32))
    rep3 = rep_flat.reshape(NW, NCHUNK, 128)
    cnt2 = (cnts[0].reshape(EP, 1), cnts[1].reshape(EP, 1))

    idx_row = _r2(rowp)
    idx_col = _r2(colp)
    idx_col_s = _r3(col_s)
    idx_crow = _r2(crow)
    idx_ccol = _r2(ccol)
    zeros_f = jnp.zeros((ACC_F, HID), f32)
    zeros_c = jnp.zeros((ACC_C, HID), f32)
    zeros_ce = jnp.zeros((CE_ACC, 16), f32)

    x0 = _enc(nap, p['node_enc'], block=2048)
    e0 = _enc(eap, p['edge_enc'], block=2048)

    lp = p['down'][0]
    gxr, gxc = _sc_gather_pair(x0, idx_row, idx_col)
    e1 = _edge_mlp(gxr, gxc, e0, lp['edge'])
    parts = _sc_scatter_fine(e1, idx_col_s, zeros_f)
    x1 = _node_mlp(x0, parts[0, :NP], parts[1, :NP], lp['node'], block=2048)

    cx = _pool(x1.reshape(NCP, 2, HID))
    ce_sums = _sc_ce_scatter(e1, rep3, zeros_ce)

    lp = p['bottleneck'][0]
    cgr, cgc = _sc_gather_pair(cx, idx_crow, idx_ccol)
    ec = _edge_mlp(cgr, cgc, ce_sums, lp['edge'], cnt=cnt2)
    cparts = _sc_scatter_coarse(ec, idx_ccol_s, zeros_c)
    cx = _node_mlp(cx, cparts[0, :NCP], cparts[1, :NCP], lp['node'], block=1024)

    lp = p['bottleneck'][1]
    cgr, cgc = _sc_gather_pair(cx, idx_crow, idx_ccol)
    ec = _edge_mlp(cgr, cgc, ec, lp['edge'])
    cparts = _sc_scatter_coarse(ec, idx_ccol_s, zeros_c)
    xup3 = _node_mlp_unpool(cx, cparts[0, :NCP], cparts[1, :NCP], lp['node'],
                            x1.reshape(NCP, 2, HID), block=1024)
    xup = xup3.reshape(NP, HID)

    lp = p['up'][0]
    gxr, gxc = _sc_gather_pair(xup, idx_row, idx_col)
    eu = _edge_mlp(gxr, gxc, e1, lp['edge'])
    parts = _sc_scatter_fine(eu, idx_col_s, zeros_f)
    out = _node_mlp_dec(xup, parts[0, :NP], parts[1, :NP], lp['node'],
                        p['dec'], block=2048)
    return out[:N_FINE]


# speedup vs baseline: 2.5242x; 1.1441x over previous
import functools

import jax
import jax.numpy as jnp
from jax import lax
from jax.experimental import pallas as pl
from jax.experimental.pallas import tpu as pltpu
from jax.experimental.pallas import tpu_sc as plsc

HID = 64
N_FINE = 10000
NP = 10240
E = 160000
EP = 163840
NC = 5000
NCP = 5120
ACC_F = 10256
ACC_C = 5136
NW = 32
EPW = EP // NW
LN_EPS = 1e-5


def _ln(h, g, be):
    mu = jnp.mean(h, axis=-1, keepdims=True)
    var = jnp.mean((h - mu) ** 2, axis=-1, keepdims=True)
    return (h - mu) * lax.rsqrt(var + LN_EPS) * g + be


def _enc(x, p, block):
    n, din = x.shape

    def body(x_ref, w1_ref, b1_ref, w2_ref, b2_ref, g_ref, be_ref, o_ref):
        h = jnp.dot(x_ref[...], w1_ref[...], preferred_element_type=jnp.float32)
        h = jnp.maximum(h + b1_ref[...], 0.0)
        h = jnp.dot(h, w2_ref[...], preferred_element_type=jnp.float32) + b2_ref[...]
        o_ref[...] = _ln(h, g_ref[...], be_ref[...])

    return pl.pallas_call(
        body,
        grid=(n // block,),
        in_specs=[
            pl.BlockSpec((block, din), lambda i: (i, 0)),
            pl.BlockSpec((din, HID), lambda i: (0, 0)),
            pl.BlockSpec((1, HID), lambda i: (0, 0)),
            pl.BlockSpec((HID, HID), lambda i: (0, 0)),
            pl.BlockSpec((1, HID), lambda i: (0, 0)),
            pl.BlockSpec((1, HID), lambda i: (0, 0)),
            pl.BlockSpec((1, HID), lambda i: (0, 0)),
        ],
        out_specs=pl.BlockSpec((block, HID), lambda i: (i, 0)),
        out_shape=jax.ShapeDtypeStruct((n, HID), jnp.float32),
    )(x, p['W1'], p['b1'].reshape(1, HID), p['W2'], p['b2'].reshape(1, HID),
      p['g'].reshape(1, HID), p['be'].reshape(1, HID))


def _edge_mlp(gxr, gxc, e, p, cnt=None, block=2048):
    n = gxr.shape[0]
    has_cnt = cnt is not None

    def body(*refs):
        if has_cnt:
            (a_ref, b_ref, e_ref, ca_ref, cb_ref, w1_ref, b1_ref, w2_ref,
             b2_ref, g_ref, be_ref, o_ref) = refs
            base = e_ref[...] * (1.0 / jnp.maximum(ca_ref[...] + cb_ref[...],
                                                   1.0))
        else:
            (a_ref, b_ref, e_ref, w1_ref, b1_ref, w2_ref, b2_ref,
             g_ref, be_ref, o_ref) = refs
            base = e_ref[...]
        w1 = w1_ref[...]
        h = (jnp.dot(a_ref[...], w1[0:HID], preferred_element_type=jnp.float32)
             + jnp.dot(b_ref[...], w1[HID:2 * HID], preferred_element_type=jnp.float32)
             + jnp.dot(base, w1[2 * HID:3 * HID], preferred_element_type=jnp.float32)
             + b1_ref[...])
        h = jnp.maximum(h, 0.0)
        h = jnp.dot(h, w2_ref[...], preferred_element_type=jnp.float32) + b2_ref[...]
        o_ref[...] = base + _ln(h, g_ref[...], be_ref[...])

    row_spec = pl.BlockSpec((block, HID), lambda i: (i, 0))
    in_specs = [row_spec, row_spec, row_spec]
    args = [gxr, gxc, e]
    if has_cnt:
        in_specs += [pl.BlockSpec((block, 1), lambda i: (i, 0)),
                     pl.BlockSpec((block, 1), lambda i: (i, 0))]
        args += [cnt[0], cnt[1]]
    in_specs += [
        pl.BlockSpec((3 * HID, HID), lambda i: (0, 0)),
        pl.BlockSpec((1, HID), lambda i: (0, 0)),
        pl.BlockSpec((HID, HID), lambda i: (0, 0)),
        pl.BlockSpec((1, HID), lambda i: (0, 0)),
        pl.BlockSpec((1, HID), lambda i: (0, 0)),
        pl.BlockSpec((1, HID), lambda i: (0, 0)),
    ]
    args += [p['W1'], p['b1'].reshape(1, HID), p['W2'], p['b2'].reshape(1, HID),
             p['g'].reshape(1, HID), p['be'].reshape(1, HID)]
    return pl.pallas_call(
        body,
        grid=(n // block,),
        in_specs=in_specs,
        out_specs=row_spec,
        out_shape=jax.ShapeDtypeStruct((n, HID), jnp.float32),
    )(*args)


def _node_mlp_body(x_ref, a_ref, b_ref, w1_ref, b1_ref, w2_ref, b2_ref,
                   g_ref, be_ref):
    agg = a_ref[...] + b_ref[...]
    w1 = w1_ref[...]
    h = (jnp.dot(x_ref[...], w1[0:HID], preferred_element_type=jnp.float32)
         + jnp.dot(agg, w1[HID:2 * HID], preferred_element_type=jnp.float32)
         + b1_ref[...])
    h = jnp.maximum(h, 0.0)
    h = jnp.dot(h, w2_ref[...], preferred_element_type=jnp.float32) + b2_ref[...]
    return x_ref[...] + _ln(h, g_ref[...], be_ref[...])


def _node_specs(block):
    row_spec = pl.BlockSpec((block, HID), lambda i: (i, 0))
    return [
        row_spec, row_spec, row_spec,
        pl.BlockSpec((2 * HID, HID), lambda i: (0, 0)),
        pl.BlockSpec((1, HID), lambda i: (0, 0)),
        pl.BlockSpec((HID, HID), lambda i: (0, 0)),
        pl.BlockSpec((1, HID), lambda i: (0, 0)),
        pl.BlockSpec((1, HID), lambda i: (0, 0)),
        pl.BlockSpec((1, HID), lambda i: (0, 0)),
    ]


def _node_args(x, agg_a, agg_b, p):
    return [x, agg_a, agg_b, p['W1'], p['b1'].reshape(1, HID), p['W2'],
            p['b2'].reshape(1, HID), p['g'].reshape(1, HID), p['be'].reshape(1, HID)]


def _node_mlp(x, agg_a, agg_b, p, block=2048):
    n = x.shape[0]

    def body(x_ref, a_ref, b_ref, w1_ref, b1_ref, w2_ref, b2_ref, g_ref,
             be_ref, o_ref):
        o_ref[...] = _node_mlp_body(x_ref, a_ref, b_ref, w1_ref, b1_ref,
                                    w2_ref, b2_ref, g_ref, be_ref)

    return pl.pallas_call(
        body,
        grid=(n // block,),
        in_specs=_node_specs(block),
        out_specs=pl.BlockSpec((block, HID), lambda i: (i, 0)),
        out_shape=jax.ShapeDtypeStruct((n, HID), jnp.float32),
    )(*_node_args(x, agg_a, agg_b, p))


def _node_mlp_unpool(x, agg_a, agg_b, p, skip3, block=1024):
    n = x.shape[0]

    def body(x_ref, a_ref, b_ref, w1_ref, b1_ref, w2_ref, b2_ref, g_ref,
             be_ref, s_ref, o_ref):
        x_new = _node_mlp_body(x_ref, a_ref, b_ref, w1_ref, b1_ref, w2_ref,
                               b2_ref, g_ref, be_ref)
        o_ref[...] = x_new[:, None, :] + s_ref[...]

    return pl.pallas_call(
        body,
        grid=(n // block,),
        in_specs=_node_specs(block) + [pl.BlockSpec((block, 2, HID), lambda i: (i, 0, 0))],
        out_specs=pl.BlockSpec((block, 2, HID), lambda i: (i, 0, 0)),
        out_shape=jax.ShapeDtypeStruct((n, 2, HID), jnp.float32),
    )(*_node_args(x, agg_a, agg_b, p), skip3)


def _node_mlp_dec(x, agg_a, agg_b, p, pdec, block=2048):
    n = x.shape[0]

    def body(x_ref, a_ref, b_ref, w1_ref, b1_ref, w2_ref, b2_ref, g_ref,
             be_ref, d1_ref, db1_ref, d2_ref, db2_ref, o_ref):
        x_new = _node_mlp_body(x_ref, a_ref, b_ref, w1_ref, b1_ref, w2_ref,
                               b2_ref, g_ref, be_ref)
        h = jnp.dot(x_new, d1_ref[...], preferred_element_type=jnp.float32)
        h = jnp.maximum(h + db1_ref[...], 0.0)
        o_ref[...] = jnp.dot(h, d2_ref[...], preferred_element_type=jnp.float32) + db2_ref[...]

    return pl.pallas_call(
        body,
        grid=(n // block,),
        in_specs=_node_specs(block) + [
            pl.BlockSpec((HID, HID), lambda i: (0, 0)),
            pl.BlockSpec((1, HID), lambda i: (0, 0)),
            pl.BlockSpec((HID, 3), lambda i: (0, 0)),
            pl.BlockSpec((1, 3), lambda i: (0, 0)),
        ],
        out_specs=pl.BlockSpec((block, 3), lambda i: (i, 0)),
        out_shape=jax.ShapeDtypeStruct((n, 3), jnp.float32),
    )(*_node_args(x, agg_a, agg_b, p), pdec['W1'], pdec['b1'].reshape(1, HID),
      pdec['W2'], pdec['b2'].reshape(1, 3))


def _pool(x3, block=1024):
    n = x3.shape[0]

    def body(x_ref, o_ref):
        o_ref[...] = 0.5 * (x_ref[:, 0, :] + x_ref[:, 1, :])

    return pl.pallas_call(
        body,
        grid=(n // block,),
        in_specs=[pl.BlockSpec((block, 2, HID), lambda i: (i, 0, 0))],
        out_specs=pl.BlockSpec((block, HID), lambda i: (i, 0)),
        out_shape=jax.ShapeDtypeStruct((n, HID), jnp.float32),
    )(x3)


@functools.cache
def _mesh():
    return plsc.VectorSubcoreMesh(core_axis_name="c", subcore_axis_name="s")


GG = 640
NGG = EPW // GG


@functools.cache
def _make_gather_pair():
    @functools.partial(
        pl.kernel, mesh=_mesh(), name="sc_gather_pair",
        out_type=[jax.ShapeDtypeStruct((EP, HID), jnp.float32),
                  jax.ShapeDtypeStruct((EP, HID), jnp.float32)],
        scratch_types=[pltpu.VMEM((EPW,), jnp.int32),
                       pltpu.VMEM((3 * GG, HID), jnp.float32),
                       pltpu.SemaphoreType.DMA,
                       pltpu.SemaphoreType.DMA],
        compiler_params=pltpu.CompilerParams(use_tc_tiling_on_sc=False),
    )
    def k(x_hbm, idxa_hbm, idxb_hbm, outa_hbm, outb_hbm, idx_v, rows_v, sem,
          wsem):
        wid = lax.axis_index("s") * 2 + lax.axis_index("c")
        base = wid * EPW

        for idx_hbm, out_hbm in ((idxa_hbm, outa_hbm), (idxb_hbm, outb_hbm)):
            pltpu.sync_copy(idx_hbm.at[wid], idx_v)
            gh = [None] * NGG
            wh = [None] * NGG
            for g in range(NGG):
                if g >= 3:
                    wh[g - 3].wait()
                gh[g] = pltpu.async_copy(
                    x_hbm.at[idx_v.at[pl.ds(g * GG, GG)]],
                    rows_v.at[pl.ds((g % 3) * GG, GG)], sem)
                if g >= 2:
                    gh[g - 2].wait()
                    wh[g - 2] = pltpu.async_copy(
                        rows_v.at[pl.ds(((g - 2) % 3) * GG, GG)],
                        out_hbm.at[pl.ds(base + (g - 2) * GG, GG)], wsem)
            for g in (NGG - 2, NGG - 1):
                gh[g].wait()
                wh[g] = pltpu.async_copy(
                    rows_v.at[pl.ds((g % 3) * GG, GG)],
                    out_hbm.at[pl.ds(base + g * GG, GG)], wsem)
            for g in range(NGG - 3, NGG):
                wh[g].wait()

    return k


def _sc_gather_pair(x, idxa, idxb):
    return _make_gather_pair()(x, idxa, idxb)


@functools.cache
def _make_scatter(acc_rows):
    rpt = acc_rows // 16
    ngs = EPW // 512

    @functools.partial(
        pl.kernel, mesh=_mesh(), name=f"sc_scatter_{acc_rows}",
        out_type=jax.ShapeDtypeStruct((2, acc_rows, HID), jnp.float32),
        scratch_types=[pltpu.VMEM_SHARED((acc_rows, HID), jnp.float32),
                       pltpu.VMEM((EPW // 128, 128), jnp.int32),
                       pltpu.VMEM((1024, HID), jnp.float32),
                       pltpu.SemaphoreType.DMA,
                       pltpu.SemaphoreType.DMA],
        compiler_params=pltpu.CompilerParams(use_tc_tiling_on_sc=False),
    )
    def k(vals_hbm, idx_hbm, zeros_hbm, out_hbm, acc_s, idx_v, ebuf,
          lsem, ssem):
        cid = lax.axis_index("c")
        sid = lax.axis_index("s")
        wid = sid * 2 + cid
        base = wid * EPW

        def fire_load(g, b):
            pltpu.async_copy(vals_hbm.at[pl.ds(base + g * 512, 512)],
                             ebuf.at[pl.ds(b * 512, 512)], lsem)

        def drain_load(b):
            pltpu.make_async_copy(vals_hbm.at[pl.ds(base, 512)],
                                  ebuf.at[pl.ds(b * 512, 512)], lsem).wait()

        def drain_adds(b):
            pltpu.make_async_copy(vals_hbm.at[pl.ds(base, 512)],
                                  ebuf.at[pl.ds(b * 512, 512)], ssem).wait()

        pltpu.sync_copy(idx_hbm.at[wid], idx_v)
        fire_load(0, 0)
        pltpu.sync_copy(zeros_hbm.at[pl.ds(sid * rpt, rpt)],
                        acc_s.at[pl.ds(sid * rpt, rpt)])
        plsc.subcore_barrier()

        def g_body(g, carry):
            b = lax.rem(g, 2)
            fire_load(g + 1, 1 - b)
            drain_load(b)
            for j in range(4):
                pltpu.async_copy(ebuf.at[pl.ds(b * 512 + j * 128, 128)],
                                 acc_s.at[idx_v.at[g * 4 + j]], ssem,
                                 add=True)
            drain_adds(b)
            return carry

        lax.fori_loop(0, ngs - 1, g_body, 0)
        b = (ngs - 1) % 2
        drain_load(b)
        for j in range(4):
            pltpu.async_copy(ebuf.at[pl.ds(b * 512 + j * 128, 128)],
                             acc_s.at[idx_v.at[(ngs - 1) * 4 + j]], ssem,
                             add=True)
        drain_adds(b)
        plsc.subcore_barrier()
        pltpu.sync_copy(acc_s.at[pl.ds(sid * rpt, rpt)],
                        out_hbm.at[cid, pl.ds(sid * rpt, rpt)])

    return k


def _sc_scatter_fine(vals, idx, zeros):
    return _make_scatter(ACC_F)(vals, idx, zeros)


def _sc_scatter_coarse(vals, idx, zeros):
    return _make_scatter(ACC_C)(vals, idx, zeros)


NKEY = NC * NC + 1
NCHUNK = EPW // 128


@functools.cache
def _make_dedup_scatter():
    @functools.partial(
        pl.kernel, mesh=_mesh(), name="sc_dedup_scatter",
        out_type=jax.ShapeDtypeStruct((NKEY, 16), jnp.int32),
        scratch_types=[pltpu.VMEM((NCHUNK, 128), jnp.int32),
                       pltpu.VMEM((EPW, 16), jnp.int32),
                       pltpu.SemaphoreType.DMA],
        compiler_params=pltpu.CompilerParams(use_tc_tiling_on_sc=False),
    )
    def k(keys_hbm, ids_hbm, table_hbm, keys_v, ids_v, sem):
        wid = lax.axis_index("s") * 2 + lax.axis_index("c")
        pltpu.sync_copy(keys_hbm.at[wid], keys_v)
        pltpu.sync_copy(ids_hbm.at[wid], ids_v)

        def body(g, c):
            for j in range(4):
                pltpu.async_copy(ids_v.at[pl.ds((g * 4 + j) * 128, 128)],
                                 table_hbm.at[keys_v.at[g * 4 + j]], sem)
            for j in range(4):
                pltpu.make_async_copy(ids_v.at[pl.ds(j * 128, 128)],
                                      table_hbm.at[keys_v.at[g * 4 + j]],
                                      sem).wait()
            return c

        lax.fori_loop(0, NCHUNK // 4, body, 0)

    return k


def _sc_dedup_scatter(keys3, ids3):
    return _make_dedup_scatter()(keys3, ids3)


@functools.cache
def _make_dedup_gather():
    @functools.partial(
        pl.kernel, mesh=_mesh(),
        out_type=[jax.ShapeDtypeStruct((NW, EPW), jnp.int32),
                  jax.ShapeDtypeStruct((NW, NCHUNK, 128), jnp.int32),
                  jax.ShapeDtypeStruct((2, EP), jnp.float32)],
        scratch_types=[pltpu.VMEM((EPW,), jnp.int32),
                       pltpu.VMEM((NCHUNK, 128), jnp.int32),
                       pltpu.VMEM((NCHUNK, 128), jnp.int32),
                       pltpu.VMEM((NCHUNK, 128), jnp.int32),
                       pltpu.VMEM((EPW,), jnp.int32),
                       pltpu.VMEM((128,), jnp.float32),
                       pltpu.VMEM_SHARED((EP,), jnp.float32),
                       pltpu.SemaphoreType.DMA,
                       pltpu.SemaphoreType.DMA],
        compiler_params=pltpu.CompilerParams(use_tc_tiling_on_sc=False),
    )
    def k(keys_hbm, ccol_hbm, table_hbm, zeros_hbm, rep_hbm, ccols_hbm,
          cnt_hbm, keys_v, ccol_v, repidx_v, ccs_v, rep_v, ones_v, acc_s,
          gsem, ssem):
        cid = lax.axis_index("c")
        sid = lax.axis_index("s")
        wid = sid * 2 + cid
        rpt = EP // 16
        pltpu.sync_copy(keys_hbm.at[wid], keys_v)
        pltpu.sync_copy(ccol_hbm.at[wid], ccol_v)
        for g in range(EPW // 1024):
            pltpu.async_copy(table_hbm.at[keys_v.at[pl.ds(g * 1024, 1024)]],
                             rep_v.at[pl.ds(g * 1024, 1024)], gsem)
        pltpu.sync_copy(zeros_hbm.at[pl.ds(sid * rpt, rpt)],
                        acc_s.at[pl.ds(sid * rpt, rpt)])
        for kk in range(8):
            ones_v[pl.ds(kk * 16, 16)] = jnp.full((16,), 1.0, jnp.float32)
        for g in range(EPW // 1024):
            pltpu.make_async_copy(table_hbm.at[keys_v.at[pl.ds(0, 1024)]],
                                  rep_v.at[pl.ds(g * 1024, 1024)], gsem).wait()
        plsc.subcore_barrier()
        base = wid * EPW
        lane = lax.iota(jnp.int32, 16)
        for j in range(NCHUNK):
            for kk in range(8):
                rep = rep_v[pl.ds(j * 128 + kk * 16, 16)]
                ids = base + j * 128 + kk * 16 + lane
                m = (rep == ids) & (ids < E)
                repidx_v[j, pl.ds(kk * 16, 16)] = rep
                ccs_v[j, pl.ds(kk * 16, 16)] = jnp.where(
                    m, ccol_v[j, pl.ds(kk * 16, 16)], NCP)
        pltpu.sync_copy(rep_v, rep_hbm.at[wid])
        pltpu.sync_copy(ccs_v, ccols_hbm.at[wid])

        def cnt_body(g, c):
            for j in range(4):
                pltpu.async_copy(ones_v, acc_s.at[repidx_v.at[g * 4 + j]],
                                 ssem, add=True)
            for j in range(4):
                pltpu.make_async_copy(ones_v,
                                      acc_s.at[repidx_v.at[g * 4 + j]],
                                      ssem).wait()
            return c

        lax.fori_loop(0, NCHUNK // 4, cnt_body, 0)
        plsc.subcore_barrier()
        pltpu.sync_copy(acc_s.at[pl.ds(sid * rpt, rpt)],
                        cnt_hbm.at[cid, pl.ds(sid * rpt, rpt)])

    return k


def _sc_dedup_gather(keys3, ccol3, table, zeros_cnt):
    return _make_dedup_gather()(keys3, ccol3, table, zeros_cnt)


CE_HALF = EP // 2
CE_ACC = CE_HALF + 128
CE_RPT = CE_ACC // 16
EPT = EP // 16


@functools.cache
def _make_ce_scatter():
    @functools.partial(
        pl.kernel, mesh=_mesh(), name="sc_ce_scatter",
        out_type=jax.ShapeDtypeStruct((EP, HID), jnp.float32),
        scratch_types=[pltpu.VMEM((EPT // 128, 128), jnp.int32),
                       pltpu.VMEM((1024, 16), jnp.float32),
                       pltpu.VMEM_SHARED((CE_ACC, 16), jnp.float32),
                       pltpu.SemaphoreType.DMA,
                       pltpu.SemaphoreType.DMA],
        compiler_params=pltpu.CompilerParams(use_tc_tiling_on_sc=False),
    )
    def k(vals_hbm, rep_hbm, zeros_hbm, out_hbm, idx_v, ebuf, acc_s,
          lsem, ssem):
        cid = lax.axis_index("c")
        sid = lax.axis_index("s")
        tbase = sid * EPT
        ngs = EPT // 512
        pltpu.sync_copy(rep_hbm.at[2 * sid], idx_v.at[pl.ds(0, NCHUNK)])
        pltpu.sync_copy(rep_hbm.at[2 * sid + 1],
                        idx_v.at[pl.ds(NCHUNK, NCHUNK)])
        lo = cid * CE_HALF

        def loc_body(j, c):
            for kk in range(8):
                v = idx_v[j, pl.ds(kk * 16, 16)]
                il = v - lo
                m = (il >= 0) & (il < CE_HALF)
                idx_v[j, pl.ds(kk * 16, 16)] = jnp.where(m, il, CE_HALF)
            return c

        lax.fori_loop(0, EPT // 128, loc_body, 0)
        for p in range(4):

            def fire_load(g, b):
                pltpu.async_copy(
                    vals_hbm.at[pl.ds(tbase + g * 512, 512),
                                pl.ds(p * 16, 16)],
                    ebuf.at[pl.ds(b * 512, 512)], lsem)

            def drain(b, sem):
                pltpu.make_async_copy(
                    vals_hbm.at[pl.ds(tbase, 512), pl.ds(p * 16, 16)],
                    ebuf.at[pl.ds(b * 512, 512)], sem).wait()

            def adds(g, b):
                for j4 in range(4):
                    pltpu.async_copy(ebuf.at[pl.ds(b * 512 + j4 * 128, 128)],
                                     acc_s.at[idx_v.at[g * 4 + j4]], ssem,
                                     add=True)
                drain(b, ssem)

            fire_load(0, 0)
            pltpu.sync_copy(zeros_hbm.at[pl.ds(sid * CE_RPT, CE_RPT)],
                            acc_s.at[pl.ds(sid * CE_RPT, CE_RPT)])
            plsc.subcore_barrier()

            def body(g, c):
                b = lax.rem(g, 2)
                fire_load(g + 1, 1 - b)
                drain(b, lsem)
                adds(g, b)
                return c

            lax.fori_loop(0, ngs - 1, body, 0)
            b = (ngs - 1) % 2
            drain(b, lsem)
            adds(ngs - 1, b)
            plsc.subcore_barrier()
            pltpu.sync_copy(
                acc_s.at[pl.ds(sid * (CE_HALF // 16), CE_HALF // 16)],
                out_hbm.at[pl.ds(cid * CE_HALF + sid * (CE_HALF // 16),
                                 CE_HALF // 16), pl.ds(p * 16, 16)])
            plsc.subcore_barrier()

    return k


def _sc_ce_scatter(vals, rep3, zeros):
    return _make_ce_scatter()(vals, rep3, zeros)


def _r3(a):
    return a.astype(jnp.int32).reshape(NW, EPW // 128, 128)


def _r2(a):
    return a.astype(jnp.int32).reshape(NW, EPW)


def kernel(node_attr, edge_attr, edge_index, params):
    f32 = jnp.float32
    p = params
    eid = jnp.arange(EP, dtype=jnp.int32)
    real = eid < E

    nap = jnp.pad(node_attr.astype(f32), ((0, NP - N_FINE), (0, 0)))
    eap = jnp.pad(edge_attr.astype(f32), ((0, EP - E), (0, 0)))
    rowp = jnp.pad(edge_index[0].astype(jnp.int32), (0, EP - E))
    colp = jnp.pad(edge_index[1].astype(jnp.int32), (0, EP - E))
    col_s = jnp.where(real, colp, NP)
    crow = rowp // 2
    ccol = colp // 2

    keys = jnp.where(real, crow * NC + ccol, NC * NC)
    ids16 = jnp.broadcast_to(eid[:, None], (EP, 16)).reshape(NW, EPW, 16)
    table = _sc_dedup_scatter(_r3(keys), ids16)
    rep_flat, idx_ccol_s, cnts = _sc_dedup_gather(
        _r2(keys * 16), _r3(ccol), table.reshape(NKEY * 16),
        jnp.zeros((EP,), f32))
    rep3 = rep_flat.reshape(NW, NCHUNK, 128)
    cnt2 = (cnts[0].reshape(EP, 1), cnts[1].reshape(EP, 1))

    idx_row = _r2(rowp)
    idx_col = _r2(colp)
    idx_col_s = _r3(col_s)
    idx_crow = _r2(crow)
    idx_ccol = _r2(ccol)
    zeros_f = jnp.zeros((ACC_F, HID), f32)
    zeros_c = jnp.zeros((ACC_C, HID), f32)
    zeros_ce = jnp.zeros((CE_ACC, 16), f32)

    x0 = _enc(nap, p['node_enc'], block=2048)
    e0 = _enc(eap, p['edge_enc'], block=2048)

    lp = p['down'][0]
    gxr, gxc = _sc_gather_pair(x0, idx_row, idx_col)
    e1 = _edge_mlp(gxr, gxc, e0, lp['edge'])
    parts = _sc_scatter_fine(e1, idx_col_s, zeros_f)
    x1 = _node_mlp(x0, parts[0, :NP], parts[1, :NP], lp['node'], block=2048)

    cx = _pool(x1.reshape(NCP, 2, HID))
    ce_sums = _sc_ce_scatter(e1, rep3, zeros_ce)

    lp = p['bottleneck'][0]
    cgr, cgc = _sc_gather_pair(cx, idx_crow, idx_ccol)
    ec = _edge_mlp(cgr, cgc, ce_sums, lp['edge'], cnt=cnt2)
    cparts = _sc_scatter_coarse(ec, idx_ccol_s, zeros_c)
    cx = _node_mlp(cx, cparts[0, :NCP], cparts[1, :NCP], lp['node'], block=1024)

    lp = p['bottleneck'][1]
    cgr, cgc = _sc_gather_pair(cx, idx_crow, idx_ccol)
    ec = _edge_mlp(cgr, cgc, ec, lp['edge'])
    cparts = _sc_scatter_coarse(ec, idx_ccol_s, zeros_c)
    xup3 = _node_mlp_unpool(cx, cparts[0, :NCP], cparts[1, :NCP], lp['node'],
                            x1.reshape(NCP, 2, HID), block=1024)
    xup = xup3.reshape(NP, HID)

    lp = p['up'][0]
    gxr, gxc = _sc_gather_pair(xup, idx_row, idx_col)
    eu = _edge_mlp(gxr, gxc, e1, lp['edge'])
    parts = _sc_scatter_fine(eu, idx_col_s, zeros_f)
    out = _node_mlp_dec(xup, parts[0, :NP], parts[1, :NP], lp['node'],
                        p['dec'], block=2048)
    return out[:N_FINE]
